# baseline (speedup 1.0000x reference)
import functools

import jax
import jax.numpy as jnp
from jax import lax
from jax.experimental import pallas as pl
from jax.experimental.pallas import tpu as pltpu
from jax.experimental.pallas import tpu_sc as plsc

N_NODES = 10000
HID = 256
BATCH = 1024
SEQL = 50
RDIM = 768
E_PAD = 163840
N_ACC = 10112
CHUNK = 128
NSUB = 16
CPS = E_PAD // NSUB // CHUNK
GRP = 40
ZROWS = N_ACC // NSUB

BB = 8
ROWS = BB * SEQL

_INTERPRET = False


def _pack_bf16(x):
    u = lax.bitcast_convert_type(x, jnp.uint32)
    r = (u + jnp.uint32(0x8000)) >> 16
    n = x.shape[-1] // 2
    return lax.bitcast_convert_type(r[..., :n] | (r[..., n:] << 16), jnp.int32)


def _unpack_bf16(w):
    u = lax.bitcast_convert_type(w, jnp.uint32)
    lo = lax.bitcast_convert_type(u << 16, jnp.float32)
    hi = lax.bitcast_convert_type(u & jnp.uint32(0xFFFF0000), jnp.float32)
    return jnp.concatenate([lo, hi], axis=-1)


def _segsum_body(table_ref, srcx_ref, dst_ref, zeros_ref, out_ref,
                 acc, src_t, dst_t, rows0, rows1, semg0, semg1, sems0, sems1):
    c = lax.axis_index("c")
    s = lax.axis_index("s")
    pltpu.sync_copy(zeros_ref, acc.at[pl.ds(s * ZROWS, ZROWS)])
    plsc.subcore_barrier()
    row0 = s * CPS

    def step(jj, _):
        j0 = jj * 2
        j1 = j0 + 1
        pltpu.make_async_copy(table_ref.at[src_t.at[j0]], rows0, semg0).wait()
        pltpu.async_copy(rows0, acc.at[dst_t.at[j0]], sems0, add=True)
        pltpu.make_async_copy(table_ref.at[src_t.at[j1]], rows1, semg1).wait()
        pltpu.async_copy(rows1, acc.at[dst_t.at[j1]], sems1, add=True)
        pltpu.make_async_copy(rows0, acc.at[dst_t.at[j0]], sems0).wait()

        @pl.when(jj < GRP // 2 - 1)
        def _():
            pltpu.async_copy(table_ref.at[src_t.at[j0 + 2]], rows0, semg0)

        pltpu.make_async_copy(rows1, acc.at[dst_t.at[j1]], sems1).wait()

        @pl.when(jj < GRP // 2 - 1)
        def _():
            pltpu.async_copy(table_ref.at[src_t.at[j1 + 2]], rows1, semg1)

        return 0

    for g in range(CPS // GRP):
        base = row0 + g * GRP
        pltpu.sync_copy(srcx_ref.at[c].at[pl.ds(base, GRP)], src_t)
        pltpu.sync_copy(dst_ref.at[pl.ds(base, GRP)], dst_t)
        pltpu.async_copy(table_ref.at[src_t.at[0]], rows0, semg0)
        pltpu.async_copy(table_ref.at[src_t.at[1]], rows1, semg1)
        lax.fori_loop(0, GRP // 2, step, 0)

    plsc.subcore_barrier()
    pltpu.sync_copy(acc.at[pl.ds(s * ZROWS, ZROWS)],
                    out_ref.at[c].at[pl.ds(s * ZROWS, ZROWS)])


def _sc_segsum(table2, srcx, dst2d, zeros):
    mesh = plsc.VectorSubcoreMesh(core_axis_name="c", subcore_axis_name="s")
    return pl.kernel(
        _segsum_body,
        out_type=jax.ShapeDtypeStruct((2, N_ACC, 128), jnp.float32),
        mesh=mesh,
        scratch_types=[
            pltpu.VMEM_SHARED((N_ACC, 128), jnp.float32),
            pltpu.VMEM((GRP, CHUNK), jnp.int32),
            pltpu.VMEM((GRP, CHUNK), jnp.int32),
            pltpu.VMEM((CHUNK, 128), jnp.float32),
            pltpu.VMEM((CHUNK, 128), jnp.float32),
            pltpu.SemaphoreType.DMA,
            pltpu.SemaphoreType.DMA,
            pltpu.SemaphoreType.DMA,
            pltpu.SemaphoreType.DMA,
        ],
    )(table2, srcx, dst2d, zeros)


IPW = BATCH // 32
LPAD = 128


def _gather3_body(t0_ref, t1_ref, t2_ref, seq_ref, o0_ref, o1_ref, o2_ref,
                  idx_t, buf0, buf1, sem0, sem1):
    c = lax.axis_index("c")
    s = lax.axis_index("s")
    it0 = (s * 2 + c) * IPW
    pltpu.sync_copy(seq_ref.at[pl.ds(it0, IPW)], idx_t)

    for t_ref, o_ref in ((t0_ref, o0_ref), (t1_ref, o1_ref), (t2_ref, o2_ref)):
        pltpu.async_copy(t_ref.at[idx_t.at[0, pl.ds(0, SEQL)]], buf0, sem0)

        def step(jj, _, t_ref=t_ref, o_ref=o_ref):
            j0 = jj * 2
            j1 = j0 + 1
            pltpu.async_copy(t_ref.at[idx_t.at[j1, pl.ds(0, SEQL)]], buf1, sem1)
            pltpu.make_async_copy(t_ref.at[idx_t.at[j0, pl.ds(0, SEQL)]], buf0, sem0).wait()
            pltpu.sync_copy(buf0, o_ref.at[it0 + j0])

            @pl.when(jj < IPW // 2 - 1)
            def _():
                pltpu.async_copy(t_ref.at[idx_t.at[j0 + 2, pl.ds(0, SEQL)]], buf0, sem0)

            pltpu.make_async_copy(t_ref.at[idx_t.at[j1, pl.ds(0, SEQL)]], buf1, sem1).wait()
            pltpu.sync_copy(buf1, o_ref.at[it0 + j1])
            return 0

        lax.fori_loop(0, IPW // 2, step, 0)


RPW = BATCH * SEQL // 32
GCH = 128


def _gather3f_body(t0_ref, t1_ref, t2_ref, seq_ref, o0_ref, o1_ref, o2_ref,
                   idx_t, buf0, buf1, sem0, sem1):
    c = lax.axis_index("c")
    s = lax.axis_index("s")
    base = (s * 2 + c) * RPW
    pltpu.sync_copy(seq_ref.at[pl.ds(base, RPW)], idx_t)

    for t_ref, o_ref in ((t0_ref, o0_ref), (t1_ref, o1_ref), (t2_ref, o2_ref)):
        pltpu.async_copy(t_ref.at[idx_t.at[pl.ds(0, GCH)]], buf0, sem0)

        def step(jj, _, t_ref=t_ref, o_ref=o_ref):
            j0 = jj * 2
            j1 = j0 + 1
            pltpu.async_copy(t_ref.at[idx_t.at[pl.ds(j1 * GCH, GCH)]], buf1, sem1)
            pltpu.make_async_copy(t_ref.at[idx_t.at[pl.ds(j0 * GCH, GCH)]], buf0, sem0).wait()
            pltpu.sync_copy(buf0, o_ref.at[pl.ds(base + j0 * GCH, GCH)])

            @pl.when(jj < RPW // GCH // 2 - 1)
            def _():
                pltpu.async_copy(t_ref.at[idx_t.at[pl.ds((j0 + 2) * GCH, GCH)]], buf0, sem0)

            pltpu.make_async_copy(t_ref.at[idx_t.at[pl.ds(j1 * GCH, GCH)]], buf1, sem1).wait()
            pltpu.sync_copy(buf1, o_ref.at[pl.ds(base + j1 * GCH, GCH)])
            return 0

        lax.fori_loop(0, RPW // GCH // 2, step, 0)
        cp = pltpu.async_copy(t_ref.at[idx_t.at[pl.ds(1536, 64)]],
                              buf0.at[pl.ds(0, 64)], sem0)
        cp.wait()
        pltpu.sync_copy(buf0.at[pl.ds(0, 64)], o_ref.at[pl.ds(base + 1536, 64)])


def _sc_gather3f(t0, t1, t2, seq_flat):
    mesh = plsc.VectorSubcoreMesh(core_axis_name="c", subcore_axis_name="s")
    osd = jax.ShapeDtypeStruct((BATCH * SEQL, HID // 2), jnp.int32)
    return pl.kernel(
        _gather3f_body,
        out_type=(osd, osd, osd),
        mesh=mesh,
        scratch_types=[
            pltpu.VMEM((RPW,), jnp.int32),
            pltpu.VMEM((GCH, HID // 2), jnp.int32),
            pltpu.VMEM((GCH, HID // 2), jnp.int32),
            pltpu.SemaphoreType.DMA,
            pltpu.SemaphoreType.DMA,
        ],
    )(t0, t1, t2, seq_flat)


def _sc_gather3(t0, t1, t2, seq_pad):
    mesh = plsc.VectorSubcoreMesh(core_axis_name="c", subcore_axis_name="s")
    osd = jax.ShapeDtypeStruct((BATCH, SEQL, HID), jnp.float32)
    return pl.kernel(
        _gather3_body,
        out_type=(osd, osd, osd),
        mesh=mesh,
        scratch_types=[
            pltpu.VMEM((IPW, LPAD), jnp.int32),
            pltpu.VMEM((SEQL, HID), jnp.float32),
            pltpu.VMEM((SEQL, HID), jnp.float32),
            pltpu.SemaphoreType.DMA,
            pltpu.SemaphoreType.DMA,
        ],
    )(t0, t1, t2, seq_pad)


def _mm_body(a_ref, w_ref, o_ref, *, relu):
    a = a_ref[...].astype(jnp.bfloat16)
    w = w_ref[...]
    y = (jnp.dot(a[0], w[:128], preferred_element_type=jnp.float32)
         + jnp.dot(a[1], w[128:], preferred_element_type=jnp.float32))
    if relu:
        y = jnp.maximum(y, 0.0)
    if o_ref.dtype == jnp.int32:
        o_ref[...] = _pack_bf16(y)
    else:
        o_ref[...] = y


def _gcn_matmul(a, w, relu, pack=False):
    rb = 400
    grid = N_NODES // rb
    ow = HID // 2 if pack else HID
    odt = jnp.int32 if pack else jnp.float32
    return pl.pallas_call(
        functools.partial(_mm_body, relu=relu),
        grid=(grid,),
        in_specs=[
            pl.BlockSpec((2, rb, 128), lambda i: (0, i, 0)),
            pl.BlockSpec((HID, HID), lambda i: (0, 0)),
        ],
        out_specs=pl.BlockSpec((rb, ow), lambda i: (i, 0)),
        out_shape=jax.ShapeDtypeStruct((N_NODES, ow), odt),
        interpret=_INTERPRET,
    )(a, w)


def _rproj_body(r_ref, w_ref, b_ref, o_ref):
    rows = [jnp.dot(r_ref[b].astype(jnp.bfloat16), w_ref[...],
                    preferred_element_type=jnp.float32)
            for b in range(BB)]
    o_ref[...] = _pack_bf16(jnp.concatenate(rows, axis=0) + b_ref[...])


def _review_proj(review, W_bre, b_bre):
    grid = BATCH // BB
    return pl.pallas_call(
        _rproj_body,
        grid=(grid,),
        in_specs=[
            pl.BlockSpec((BB, SEQL, RDIM), lambda i: (i, 0, 0)),
            pl.BlockSpec((RDIM, HID), lambda i: (0, 0)),
            pl.BlockSpec((1, HID), lambda i: (0, 0)),
        ],
        out_specs=pl.BlockSpec((ROWS, HID // 2), lambda i: (i, 0)),
        out_shape=jax.ShapeDtypeStruct((BATCH * SEQL, HID // 2), jnp.int32),
        interpret=_INTERPRET,
    )(review, W_bre, b_bre.reshape(1, HID))


def _fused_body(g_ref, h_ref, p_ref, r_ref, m_ref, A_ref, AT_ref,
                wbre_ref, bbre_ref, w1_ref, b1_ref, w2_ref, b2_ref,
                w3_ref, b3_ref, w4_ref, b4_ref, o_ref):
    f32 = jnp.float32
    bf16 = jnp.bfloat16
    g = _unpack_bf16(g_ref[...])
    h = _unpack_bf16(h_ref[...])
    p = _unpack_bf16(p_ref[...])
    rp = jnp.concatenate(
        [jnp.dot(r_ref[b].astype(bf16), wbre_ref[...],
                 preferred_element_type=f32) for b in range(BB)],
        axis=0) + bbre_ref[...]
    A = A_ref[...]
    AT = AT_ref[...]

    zcat = jnp.concatenate([g, rp, p], axis=1).astype(bf16)
    z = jnp.tanh(jnp.dot(zcat, w1_ref[...], preferred_element_type=f32) + b1_ref[...])

    seqlen = jnp.sum(m_ref[...], axis=1)
    s = jnp.dot(A, g, preferred_element_type=f32) / seqlen[:, None]

    zproj = jnp.dot(z.astype(bf16), w2_ref[...], preferred_element_type=f32) + b2_ref[...]
    sproj = jnp.dot(s, w3_ref[...], preferred_element_type=f32) + b3_ref[...]
    gate = jax.nn.sigmoid(zproj + jnp.dot(AT, sproj, preferred_element_type=f32))
    beta = jnp.dot(gate.astype(bf16), w4_ref[...], preferred_element_type=f32) + b4_ref[...]

    S = jnp.dot(A, beta * g, preferred_element_type=f32)
    t = jnp.dot(AT, S, preferred_element_type=f32) * h
    e = jnp.exp(t)
    denom = jnp.dot(A, e, preferred_element_type=f32)
    o_ref[...] = e * jnp.dot(AT, 1.0 / denom, preferred_element_type=f32)


def _fused_gating(g, h, p, review, mask, A, AT,
                  W_bre, b_bre, W1, b1, W2, b2, W3, b3, W4, b4):
    grid = BATCH // BB
    row_spec = pl.BlockSpec((ROWS, HID // 2), lambda i: (i, 0))
    full = lambda shape: pl.BlockSpec(shape, lambda i: (0, 0))
    return pl.pallas_call(
        _fused_body,
        grid=(grid,),
        in_specs=[
            row_spec,
            row_spec,
            row_spec,
            pl.BlockSpec((BB, SEQL, RDIM), lambda i: (i, 0, 0)),
            pl.BlockSpec((BB, SEQL), lambda i: (i, 0)),
            full((BB, ROWS)),
            full((ROWS, BB)),
            full((RDIM, HID)),
            full((1, HID)),
            full((3 * HID, HID)),
            full((1, HID)),
            full((HID, HID)),
            full((1, HID)),
            full((HID, HID)),
            full((1, HID)),
            full((HID, HID)),
            full((1, HID)),
        ],
        out_specs=pl.BlockSpec((ROWS, HID), lambda i: (i, 0)),
        out_shape=jax.ShapeDtypeStruct((BATCH * SEQL, HID), jnp.float32),
        interpret=_INTERPRET,
    )(g, h, p, review, mask, A, AT,
      W_bre, b_bre.reshape(1, HID), W1, b1.reshape(1, HID),
      W2, b2.reshape(1, HID), W3, b3.reshape(1, HID), W4, b4.reshape(1, HID))


def kernel(seq, review, mask, edge_index, emb, pos_table, Wg1, Wg2,
           W_bre, b_bre, W1, b1, W2, b2, W3, b3, W4, b4):
    src = edge_index[0]
    dst = edge_index[1]

    e = src.shape[0]
    srcp = jnp.concatenate([src, jnp.zeros((E_PAD - e,), src.dtype)]).astype(jnp.int32)
    dstp = jnp.concatenate([dst, jnp.full((E_PAD - e,), N_NODES, dst.dtype)]).astype(jnp.int32)
    srcx = jnp.stack([2 * srcp, 2 * srcp + 1]).reshape(2, E_PAD // CHUNK, CHUNK)
    dst2d = dstp.reshape(E_PAD // CHUNK, CHUNK)
    zeros = jnp.zeros((ZROWS, 128), jnp.float32)

    agg1 = _sc_segsum(emb.reshape(2 * N_NODES, 128), srcx, dst2d, zeros)
    x1 = _gcn_matmul(agg1, Wg1.astype(jnp.bfloat16), relu=True)
    agg2 = _sc_segsum(x1.reshape(2 * N_NODES, 128), srcx, dst2d, zeros)
    x2 = _gcn_matmul(agg2, Wg2.astype(jnp.bfloat16), relu=False, pack=True)

    bf16 = jnp.bfloat16

    h, g, p = _sc_gather3f(_pack_bf16(emb), x2, _pack_bf16(pos_table),
                           seq.reshape(-1).astype(jnp.int32))

    cols = jnp.arange(ROWS, dtype=jnp.int32)
    rows = jnp.arange(BB, dtype=jnp.int32)
    A = (cols[None, :] // SEQL == rows[:, None]).astype(jnp.float32)
    scores = _fused_gating(g, h, p, review, mask, A, A.T,
                           W_bre.astype(bf16), b_bre,
                           W1.astype(bf16), b1, W2.astype(bf16), b2,
                           W3, b3, W4.astype(bf16), b4)
    return scores.reshape(BATCH, SEQL, HID)


# speedup vs baseline: 1.0553x; 1.0553x over previous
import functools

import jax
import jax.numpy as jnp
from jax import lax
from jax.experimental import pallas as pl
from jax.experimental.pallas import tpu as pltpu
from jax.experimental.pallas import tpu_sc as plsc

N_NODES = 10000
HID = 256
BATCH = 1024
SEQL = 50
RDIM = 768
E_PAD = 163840
N_ACC = 10112
CHUNK = 80
NSUB = 16
CPS = E_PAD // NSUB // CHUNK
GRP = 64
ZROWS = N_ACC // NSUB

BB = 8
ROWS = BB * SEQL

_INTERPRET = False


def _pack_bf16(x):
    u = lax.bitcast_convert_type(x, jnp.uint32)
    r = (u + jnp.uint32(0x8000)) >> 16
    n = x.shape[-1] // 2
    return lax.bitcast_convert_type(r[..., :n] | (r[..., n:] << 16), jnp.int32)


def _unpack_bf16(w):
    u = lax.bitcast_convert_type(w, jnp.uint32)
    lo = lax.bitcast_convert_type(u << 16, jnp.float32)
    hi = lax.bitcast_convert_type(u & jnp.uint32(0xFFFF0000), jnp.float32)
    return jnp.concatenate([lo, hi], axis=-1)


def _segsum_body(table_ref, srcx_ref, dst_ref, zeros_ref, out_ref,
                 acc, src_t, dst_t, rows0, rows1, semg0, semg1, sems0, sems1):
    c = lax.axis_index("c")
    s = lax.axis_index("s")
    pltpu.sync_copy(zeros_ref, acc.at[pl.ds(s * ZROWS, ZROWS)])
    plsc.subcore_barrier()
    row0 = s * CPS

    def step(jj, _):
        j0 = jj * 2
        j1 = j0 + 1
        pltpu.async_copy(table_ref.at[src_t.at[j1]], rows1, semg1)
        pltpu.make_async_copy(table_ref.at[src_t.at[j0]], rows0, semg0).wait()
        pltpu.sync_copy(rows0, acc.at[dst_t.at[j0]], add=True)

        @pl.when(jj < GRP // 2 - 1)
        def _():
            pltpu.async_copy(table_ref.at[src_t.at[j0 + 2]], rows0, semg0)

        pltpu.make_async_copy(table_ref.at[src_t.at[j1]], rows1, semg1).wait()
        pltpu.sync_copy(rows1, acc.at[dst_t.at[j1]], add=True)
        return 0

    for g in range(CPS // GRP):
        base = row0 + g * GRP
        pltpu.sync_copy(srcx_ref.at[c].at[pl.ds(base, GRP)], src_t)
        pltpu.sync_copy(dst_ref.at[pl.ds(base, GRP)], dst_t)
        pltpu.async_copy(table_ref.at[src_t.at[0]], rows0, semg0)
        lax.fori_loop(0, GRP // 2, step, 0)

    plsc.subcore_barrier()
    pltpu.sync_copy(acc.at[pl.ds(s * ZROWS, ZROWS)],
                    out_ref.at[c].at[pl.ds(s * ZROWS, ZROWS)])


def _sc_segsum(table2, srcx, dst2d, zeros):
    mesh = plsc.VectorSubcoreMesh(core_axis_name="c", subcore_axis_name="s")
    return pl.kernel(
        _segsum_body,
        out_type=jax.ShapeDtypeStruct((2, N_ACC, 128), jnp.float32),
        mesh=mesh,
        scratch_types=[
            pltpu.VMEM_SHARED((N_ACC, 128), jnp.float32),
            pltpu.VMEM((GRP, CHUNK), jnp.int32),
            pltpu.VMEM((GRP, CHUNK), jnp.int32),
            pltpu.VMEM((CHUNK, 128), jnp.float32),
            pltpu.VMEM((CHUNK, 128), jnp.float32),
            pltpu.SemaphoreType.DMA,
            pltpu.SemaphoreType.DMA,
            pltpu.SemaphoreType.DMA,
            pltpu.SemaphoreType.DMA,
        ],
    )(table2, srcx, dst2d, zeros)


IPW = BATCH // 32
LPAD = 128


def _gather3_body(t0_ref, t1_ref, t2_ref, seq_ref, o0_ref, o1_ref, o2_ref,
                  idx_t, buf0, buf1, sem0, sem1):
    c = lax.axis_index("c")
    s = lax.axis_index("s")
    it0 = (s * 2 + c) * IPW
    pltpu.sync_copy(seq_ref.at[pl.ds(it0, IPW)], idx_t)

    for t_ref, o_ref in ((t0_ref, o0_ref), (t1_ref, o1_ref), (t2_ref, o2_ref)):
        pltpu.async_copy(t_ref.at[idx_t.at[0, pl.ds(0, SEQL)]], buf0, sem0)

        def step(jj, _, t_ref=t_ref, o_ref=o_ref):
            j0 = jj * 2
            j1 = j0 + 1
            pltpu.async_copy(t_ref.at[idx_t.at[j1, pl.ds(0, SEQL)]], buf1, sem1)
            pltpu.make_async_copy(t_ref.at[idx_t.at[j0, pl.ds(0, SEQL)]], buf0, sem0).wait()
            pltpu.sync_copy(buf0, o_ref.at[it0 + j0])

            @pl.when(jj < IPW // 2 - 1)
            def _():
                pltpu.async_copy(t_ref.at[idx_t.at[j0 + 2, pl.ds(0, SEQL)]], buf0, sem0)

            pltpu.make_async_copy(t_ref.at[idx_t.at[j1, pl.ds(0, SEQL)]], buf1, sem1).wait()
            pltpu.sync_copy(buf1, o_ref.at[it0 + j1])
            return 0

        lax.fori_loop(0, IPW // 2, step, 0)


RPW = BATCH * SEQL // 32
GCH = 128


def _gather3f_body(t0_ref, t1_ref, t2_ref, seq_ref, o0_ref, o1_ref, o2_ref,
                   idx_t, buf0, buf1, sem0, sem1):
    c = lax.axis_index("c")
    s = lax.axis_index("s")
    base = (s * 2 + c) * RPW
    pltpu.sync_copy(seq_ref.at[pl.ds(base, RPW)], idx_t)

    for t_ref, o_ref in ((t0_ref, o0_ref), (t1_ref, o1_ref), (t2_ref, o2_ref)):
        pltpu.async_copy(t_ref.at[idx_t.at[pl.ds(0, GCH)]], buf0, sem0)

        def step(jj, _, t_ref=t_ref, o_ref=o_ref):
            j0 = jj * 2
            j1 = j0 + 1
            pltpu.async_copy(t_ref.at[idx_t.at[pl.ds(j1 * GCH, GCH)]], buf1, sem1)
            pltpu.make_async_copy(t_ref.at[idx_t.at[pl.ds(j0 * GCH, GCH)]], buf0, sem0).wait()
            pltpu.sync_copy(buf0, o_ref.at[pl.ds(base + j0 * GCH, GCH)])

            @pl.when(jj < RPW // GCH // 2 - 1)
            def _():
                pltpu.async_copy(t_ref.at[idx_t.at[pl.ds((j0 + 2) * GCH, GCH)]], buf0, sem0)

            pltpu.make_async_copy(t_ref.at[idx_t.at[pl.ds(j1 * GCH, GCH)]], buf1, sem1).wait()
            pltpu.sync_copy(buf1, o_ref.at[pl.ds(base + j1 * GCH, GCH)])
            return 0

        lax.fori_loop(0, RPW // GCH // 2, step, 0)
        cp = pltpu.async_copy(t_ref.at[idx_t.at[pl.ds(1536, 64)]],
                              buf0.at[pl.ds(0, 64)], sem0)
        cp.wait()
        pltpu.sync_copy(buf0.at[pl.ds(0, 64)], o_ref.at[pl.ds(base + 1536, 64)])


def _sc_gather3f(t0, t1, t2, seq_flat):
    mesh = plsc.VectorSubcoreMesh(core_axis_name="c", subcore_axis_name="s")
    osd = jax.ShapeDtypeStruct((BATCH * SEQL, HID // 2), jnp.int32)
    return pl.kernel(
        _gather3f_body,
        out_type=(osd, osd, osd),
        mesh=mesh,
        scratch_types=[
            pltpu.VMEM((RPW,), jnp.int32),
            pltpu.VMEM((GCH, HID // 2), jnp.int32),
            pltpu.VMEM((GCH, HID // 2), jnp.int32),
            pltpu.SemaphoreType.DMA,
            pltpu.SemaphoreType.DMA,
        ],
    )(t0, t1, t2, seq_flat)


def _sc_gather3(t0, t1, t2, seq_pad):
    mesh = plsc.VectorSubcoreMesh(core_axis_name="c", subcore_axis_name="s")
    osd = jax.ShapeDtypeStruct((BATCH, SEQL, HID), jnp.float32)
    return pl.kernel(
        _gather3_body,
        out_type=(osd, osd, osd),
        mesh=mesh,
        scratch_types=[
            pltpu.VMEM((IPW, LPAD), jnp.int32),
            pltpu.VMEM((SEQL, HID), jnp.float32),
            pltpu.VMEM((SEQL, HID), jnp.float32),
            pltpu.SemaphoreType.DMA,
            pltpu.SemaphoreType.DMA,
        ],
    )(t0, t1, t2, seq_pad)


def _mm_body(a_ref, w_ref, o_ref, *, relu):
    a = a_ref[...].astype(jnp.bfloat16)
    w = w_ref[...]
    y = (jnp.dot(a[0], w[:128], preferred_element_type=jnp.float32)
         + jnp.dot(a[1], w[128:], preferred_element_type=jnp.float32))
    if relu:
        y = jnp.maximum(y, 0.0)
    if o_ref.dtype == jnp.int32:
        o_ref[...] = _pack_bf16(y)
    else:
        o_ref[...] = y


def _gcn_matmul(a, w, relu, pack=False):
    rb = 400
    grid = N_NODES // rb
    ow = HID // 2 if pack else HID
    odt = jnp.int32 if pack else jnp.float32
    return pl.pallas_call(
        functools.partial(_mm_body, relu=relu),
        grid=(grid,),
        in_specs=[
            pl.BlockSpec((2, rb, 128), lambda i: (0, i, 0)),
            pl.BlockSpec((HID, HID), lambda i: (0, 0)),
        ],
        out_specs=pl.BlockSpec((rb, ow), lambda i: (i, 0)),
        out_shape=jax.ShapeDtypeStruct((N_NODES, ow), odt),
        interpret=_INTERPRET,
    )(a, w)


def _rproj_body(r_ref, w_ref, b_ref, o_ref):
    rows = [jnp.dot(r_ref[b].astype(jnp.bfloat16), w_ref[...],
                    preferred_element_type=jnp.float32)
            for b in range(BB)]
    o_ref[...] = _pack_bf16(jnp.concatenate(rows, axis=0) + b_ref[...])


def _review_proj(review, W_bre, b_bre):
    grid = BATCH // BB
    return pl.pallas_call(
        _rproj_body,
        grid=(grid,),
        in_specs=[
            pl.BlockSpec((BB, SEQL, RDIM), lambda i: (i, 0, 0)),
            pl.BlockSpec((RDIM, HID), lambda i: (0, 0)),
            pl.BlockSpec((1, HID), lambda i: (0, 0)),
        ],
        out_specs=pl.BlockSpec((ROWS, HID // 2), lambda i: (i, 0)),
        out_shape=jax.ShapeDtypeStruct((BATCH * SEQL, HID // 2), jnp.int32),
        interpret=_INTERPRET,
    )(review, W_bre, b_bre.reshape(1, HID))


def _fused_body(g_ref, h_ref, p_ref, r_ref, m_ref, A_ref, AT_ref,
                wbre_ref, bbre_ref, w1_ref, b1_ref, w2_ref, b2_ref,
                w3_ref, b3_ref, w4_ref, b4_ref, o_ref):
    f32 = jnp.float32
    bf16 = jnp.bfloat16
    g = _unpack_bf16(g_ref[...])
    h = _unpack_bf16(h_ref[...])
    p = _unpack_bf16(p_ref[...])
    rp = jnp.concatenate(
        [jnp.dot(r_ref[b].astype(bf16), wbre_ref[...],
                 preferred_element_type=f32) for b in range(BB)],
        axis=0) + bbre_ref[...]
    A = A_ref[...]
    AT = AT_ref[...]

    zcat = jnp.concatenate([g, rp, p], axis=1).astype(bf16)
    z = jnp.tanh(jnp.dot(zcat, w1_ref[...], preferred_element_type=f32) + b1_ref[...])

    seqlen = jnp.sum(m_ref[...], axis=1)
    s = jnp.dot(A, g, preferred_element_type=f32) / seqlen[:, None]

    zproj = jnp.dot(z.astype(bf16), w2_ref[...], preferred_element_type=f32) + b2_ref[...]
    sproj = jnp.dot(s, w3_ref[...], preferred_element_type=f32) + b3_ref[...]
    gate = jax.nn.sigmoid(zproj + jnp.dot(AT, sproj, preferred_element_type=f32))
    beta = jnp.dot(gate.astype(bf16), w4_ref[...], preferred_element_type=f32) + b4_ref[...]

    S = jnp.dot(A, beta * g, preferred_element_type=f32)
    t = jnp.dot(AT, S, preferred_element_type=f32) * h
    e = jnp.exp(t)
    denom = jnp.dot(A, e, preferred_element_type=f32)
    o_ref[...] = e * jnp.dot(AT, 1.0 / denom, preferred_element_type=f32)


def _fused_gating(g, h, p, review, mask, A, AT,
                  W_bre, b_bre, W1, b1, W2, b2, W3, b3, W4, b4):
    grid = BATCH // BB
    row_spec = pl.BlockSpec((ROWS, HID // 2), lambda i: (i, 0))
    full = lambda shape: pl.BlockSpec(shape, lambda i: (0, 0))
    return pl.pallas_call(
        _fused_body,
        grid=(grid,),
        in_specs=[
            row_spec,
            row_spec,
            row_spec,
            pl.BlockSpec((BB, SEQL, RDIM), lambda i: (i, 0, 0)),
            pl.BlockSpec((BB, SEQL), lambda i: (i, 0)),
            full((BB, ROWS)),
            full((ROWS, BB)),
            full((RDIM, HID)),
            full((1, HID)),
            full((3 * HID, HID)),
            full((1, HID)),
            full((HID, HID)),
            full((1, HID)),
            full((HID, HID)),
            full((1, HID)),
            full((HID, HID)),
            full((1, HID)),
        ],
        out_specs=pl.BlockSpec((ROWS, HID), lambda i: (i, 0)),
        out_shape=jax.ShapeDtypeStruct((BATCH * SEQL, HID), jnp.float32),
        interpret=_INTERPRET,
    )(g, h, p, review, mask, A, AT,
      W_bre, b_bre.reshape(1, HID), W1, b1.reshape(1, HID),
      W2, b2.reshape(1, HID), W3, b3.reshape(1, HID), W4, b4.reshape(1, HID))


def kernel(seq, review, mask, edge_index, emb, pos_table, Wg1, Wg2,
           W_bre, b_bre, W1, b1, W2, b2, W3, b3, W4, b4):
    src = edge_index[0]
    dst = edge_index[1]

    e = src.shape[0]
    srcp = jnp.concatenate([src, jnp.zeros((E_PAD - e,), src.dtype)]).astype(jnp.int32)
    dstp = jnp.concatenate([dst, jnp.full((E_PAD - e,), N_NODES, dst.dtype)]).astype(jnp.int32)
    srcx = jnp.stack([2 * srcp, 2 * srcp + 1]).reshape(2, E_PAD // CHUNK, CHUNK)
    dst2d = dstp.reshape(E_PAD // CHUNK, CHUNK)
    zeros = jnp.zeros((ZROWS, 128), jnp.float32)

    agg1 = _sc_segsum(emb.reshape(2 * N_NODES, 128), srcx, dst2d, zeros)
    x1 = _gcn_matmul(agg1, Wg1.astype(jnp.bfloat16), relu=True)
    agg2 = _sc_segsum(x1.reshape(2 * N_NODES, 128), srcx, dst2d, zeros)
    x2 = _gcn_matmul(agg2, Wg2.astype(jnp.bfloat16), relu=False, pack=True)

    bf16 = jnp.bfloat16

    h, g, p = _sc_gather3f(_pack_bf16(emb), x2, _pack_bf16(pos_table),
                           seq.reshape(-1).astype(jnp.int32))

    cols = jnp.arange(ROWS, dtype=jnp.int32)
    rows = jnp.arange(BB, dtype=jnp.int32)
    A = (cols[None, :] // SEQL == rows[:, None]).astype(jnp.float32)
    scores = _fused_gating(g, h, p, review, mask, A, A.T,
                           W_bre.astype(bf16), b_bre,
                           W1.astype(bf16), b1, W2.astype(bf16), b2,
                           W3, b3, W4.astype(bf16), b4)
    return scores.reshape(BATCH, SEQL, HID)


# speedup vs baseline: 1.1308x; 1.0715x over previous
import functools

import jax
import jax.numpy as jnp
from jax import lax
from jax.experimental import pallas as pl
from jax.experimental.pallas import tpu as pltpu
from jax.experimental.pallas import tpu_sc as plsc

N_NODES = 10000
HID = 256
BATCH = 1024
SEQL = 50
RDIM = 768
E_PAD = 163840
N_ACC = 10112
CHUNK = 80
NSUB = 16
CPS = E_PAD // NSUB // CHUNK
GRP = 64
ZROWS = N_ACC // NSUB

BB = 8
ROWS = BB * SEQL

_INTERPRET = False


def _pack_bf16(x):
    u = lax.bitcast_convert_type(x, jnp.uint32)
    r = (u + jnp.uint32(0x8000)) >> 16
    n = x.shape[-1] // 2
    return lax.bitcast_convert_type(r[..., :n] | (r[..., n:] << 16), jnp.int32)


def _unpack_bf16(w):
    u = lax.bitcast_convert_type(w, jnp.uint32)
    lo = lax.bitcast_convert_type(u << 16, jnp.float32)
    hi = lax.bitcast_convert_type(u & jnp.uint32(0xFFFF0000), jnp.float32)
    return jnp.concatenate([lo, hi], axis=-1)


def _segsum_body(table_ref, srcx_ref, dst_ref, zeros_ref, out_ref,
                 acc, src_t, dst_t, rows0, rows1, semg0, semg1, sems0, sems1):
    c = lax.axis_index("c")
    s = lax.axis_index("s")
    pltpu.sync_copy(zeros_ref, acc.at[pl.ds(s * ZROWS, ZROWS)])
    plsc.subcore_barrier()
    row0 = s * CPS

    def step(jj, _):
        j0 = jj * 2
        j1 = j0 + 1
        pltpu.async_copy(table_ref.at[src_t.at[j1]], rows1, semg1)
        pltpu.make_async_copy(table_ref.at[src_t.at[j0]], rows0, semg0).wait()
        pltpu.sync_copy(rows0, acc.at[dst_t.at[j0]], add=True)

        @pl.when(jj < GRP // 2 - 1)
        def _():
            pltpu.async_copy(table_ref.at[src_t.at[j0 + 2]], rows0, semg0)

        pltpu.make_async_copy(table_ref.at[src_t.at[j1]], rows1, semg1).wait()
        pltpu.sync_copy(rows1, acc.at[dst_t.at[j1]], add=True)
        return 0

    for g in range(CPS // GRP):
        base = row0 + g * GRP
        pltpu.sync_copy(srcx_ref.at[c].at[pl.ds(base, GRP)], src_t)
        pltpu.sync_copy(dst_ref.at[pl.ds(base, GRP)], dst_t)
        pltpu.async_copy(table_ref.at[src_t.at[0]], rows0, semg0)
        lax.fori_loop(0, GRP // 2, step, 0)

    plsc.subcore_barrier()
    pltpu.sync_copy(acc.at[pl.ds(s * ZROWS, ZROWS)],
                    out_ref.at[c].at[pl.ds(s * ZROWS, ZROWS)])


def _sc_segsum(table2, srcx, dst2d, zeros):
    mesh = plsc.VectorSubcoreMesh(core_axis_name="c", subcore_axis_name="s")
    return pl.kernel(
        _segsum_body,
        out_type=jax.ShapeDtypeStruct((2, N_ACC, 128), jnp.float32),
        mesh=mesh,
        scratch_types=[
            pltpu.VMEM_SHARED((N_ACC, 128), jnp.float32),
            pltpu.VMEM((GRP, CHUNK), jnp.int32),
            pltpu.VMEM((GRP, CHUNK), jnp.int32),
            pltpu.VMEM((CHUNK, 128), jnp.float32),
            pltpu.VMEM((CHUNK, 128), jnp.float32),
            pltpu.SemaphoreType.DMA,
            pltpu.SemaphoreType.DMA,
            pltpu.SemaphoreType.DMA,
            pltpu.SemaphoreType.DMA,
        ],
    )(table2, srcx, dst2d, zeros)


IPW = BATCH // 32
LPAD = 128


def _gather3_body(t0_ref, t1_ref, t2_ref, seq_ref, o0_ref, o1_ref, o2_ref,
                  idx_t, buf0, buf1, sem0, sem1):
    c = lax.axis_index("c")
    s = lax.axis_index("s")
    it0 = (s * 2 + c) * IPW
    pltpu.sync_copy(seq_ref.at[pl.ds(it0, IPW)], idx_t)

    for t_ref, o_ref in ((t0_ref, o0_ref), (t1_ref, o1_ref), (t2_ref, o2_ref)):
        pltpu.async_copy(t_ref.at[idx_t.at[0, pl.ds(0, SEQL)]], buf0, sem0)

        def step(jj, _, t_ref=t_ref, o_ref=o_ref):
            j0 = jj * 2
            j1 = j0 + 1
            pltpu.async_copy(t_ref.at[idx_t.at[j1, pl.ds(0, SEQL)]], buf1, sem1)
            pltpu.make_async_copy(t_ref.at[idx_t.at[j0, pl.ds(0, SEQL)]], buf0, sem0).wait()
            pltpu.sync_copy(buf0, o_ref.at[it0 + j0])

            @pl.when(jj < IPW // 2 - 1)
            def _():
                pltpu.async_copy(t_ref.at[idx_t.at[j0 + 2, pl.ds(0, SEQL)]], buf0, sem0)

            pltpu.make_async_copy(t_ref.at[idx_t.at[j1, pl.ds(0, SEQL)]], buf1, sem1).wait()
            pltpu.sync_copy(buf1, o_ref.at[it0 + j1])
            return 0

        lax.fori_loop(0, IPW // 2, step, 0)


RPW = BATCH * SEQL // 32
GCH = 128


def _gather3f_body(t0_ref, t1_ref, t2_ref, seq_ref, o0_ref, o1_ref, o2_ref,
                   idx_t, buf0, buf1, sem0, sem1):
    c = lax.axis_index("c")
    s = lax.axis_index("s")
    base = (s * 2 + c) * RPW
    pltpu.sync_copy(seq_ref.at[pl.ds(base, RPW)], idx_t)

    for t_ref, o_ref in ((t0_ref, o0_ref), (t1_ref, o1_ref), (t2_ref, o2_ref)):
        pltpu.async_copy(t_ref.at[idx_t.at[pl.ds(0, GCH)]], buf0, sem0)

        def step(jj, _, t_ref=t_ref, o_ref=o_ref):
            j0 = jj * 2
            j1 = j0 + 1
            pltpu.async_copy(t_ref.at[idx_t.at[pl.ds(j1 * GCH, GCH)]], buf1, sem1)
            pltpu.make_async_copy(t_ref.at[idx_t.at[pl.ds(j0 * GCH, GCH)]], buf0, sem0).wait()
            pltpu.sync_copy(buf0, o_ref.at[pl.ds(base + j0 * GCH, GCH)])

            @pl.when(jj < RPW // GCH // 2 - 1)
            def _():
                pltpu.async_copy(t_ref.at[idx_t.at[pl.ds((j0 + 2) * GCH, GCH)]], buf0, sem0)

            pltpu.make_async_copy(t_ref.at[idx_t.at[pl.ds(j1 * GCH, GCH)]], buf1, sem1).wait()
            pltpu.sync_copy(buf1, o_ref.at[pl.ds(base + j1 * GCH, GCH)])
            return 0

        lax.fori_loop(0, RPW // GCH // 2, step, 0)
        cp = pltpu.async_copy(t_ref.at[idx_t.at[pl.ds(1536, 64)]],
                              buf0.at[pl.ds(0, 64)], sem0)
        cp.wait()
        pltpu.sync_copy(buf0.at[pl.ds(0, 64)], o_ref.at[pl.ds(base + 1536, 64)])


def _sc_gather3f(t0, t1, t2, seq_flat):
    mesh = plsc.VectorSubcoreMesh(core_axis_name="c", subcore_axis_name="s")
    osd = jax.ShapeDtypeStruct((BATCH * SEQL, HID // 2), jnp.int32)
    return pl.kernel(
        _gather3f_body,
        out_type=(osd, osd, osd),
        mesh=mesh,
        scratch_types=[
            pltpu.VMEM((RPW,), jnp.int32),
            pltpu.VMEM((GCH, HID // 2), jnp.int32),
            pltpu.VMEM((GCH, HID // 2), jnp.int32),
            pltpu.SemaphoreType.DMA,
            pltpu.SemaphoreType.DMA,
        ],
    )(t0, t1, t2, seq_flat)


def _sc_gather3(t0, t1, t2, seq_pad):
    mesh = plsc.VectorSubcoreMesh(core_axis_name="c", subcore_axis_name="s")
    osd = jax.ShapeDtypeStruct((BATCH, SEQL, HID), jnp.float32)
    return pl.kernel(
        _gather3_body,
        out_type=(osd, osd, osd),
        mesh=mesh,
        scratch_types=[
            pltpu.VMEM((IPW, LPAD), jnp.int32),
            pltpu.VMEM((SEQL, HID), jnp.float32),
            pltpu.VMEM((SEQL, HID), jnp.float32),
            pltpu.SemaphoreType.DMA,
            pltpu.SemaphoreType.DMA,
        ],
    )(t0, t1, t2, seq_pad)


def _mm_body(a_ref, w_ref, o_ref, *, relu):
    a = a_ref[...].astype(jnp.bfloat16)
    w = w_ref[...]
    y = (jnp.dot(a[0], w[:128], preferred_element_type=jnp.float32)
         + jnp.dot(a[1], w[128:], preferred_element_type=jnp.float32))
    if relu:
        y = jnp.maximum(y, 0.0)
    if o_ref.dtype == jnp.int32:
        o_ref[...] = _pack_bf16(y)
    else:
        o_ref[...] = y


def _gcn_matmul(a, w, relu, pack=False):
    rb = 400
    grid = N_NODES // rb
    ow = HID // 2 if pack else HID
    odt = jnp.int32 if pack else jnp.float32
    return pl.pallas_call(
        functools.partial(_mm_body, relu=relu),
        grid=(grid,),
        in_specs=[
            pl.BlockSpec((2, rb, 128), lambda i: (0, i, 0)),
            pl.BlockSpec((HID, HID), lambda i: (0, 0)),
        ],
        out_specs=pl.BlockSpec((rb, ow), lambda i: (i, 0)),
        out_shape=jax.ShapeDtypeStruct((N_NODES, ow), odt),
        interpret=_INTERPRET,
    )(a, w)


def _rproj_body(r_ref, w_ref, b_ref, o_ref):
    rows = [jnp.dot(r_ref[b].astype(jnp.bfloat16), w_ref[...],
                    preferred_element_type=jnp.float32)
            for b in range(BB)]
    o_ref[...] = _pack_bf16(jnp.concatenate(rows, axis=0) + b_ref[...])


def _review_proj(review, W_bre, b_bre):
    grid = BATCH // BB
    return pl.pallas_call(
        _rproj_body,
        grid=(grid,),
        in_specs=[
            pl.BlockSpec((BB, SEQL, RDIM), lambda i: (i, 0, 0)),
            pl.BlockSpec((RDIM, HID), lambda i: (0, 0)),
            pl.BlockSpec((1, HID), lambda i: (0, 0)),
        ],
        out_specs=pl.BlockSpec((ROWS, HID // 2), lambda i: (i, 0)),
        out_shape=jax.ShapeDtypeStruct((BATCH * SEQL, HID // 2), jnp.int32),
        interpret=_INTERPRET,
    )(review, W_bre, b_bre.reshape(1, HID))


def _fused_body(g_ref, h_ref, p_ref, r_ref, m_ref, A_ref, AT_ref,
                wbre_ref, bbre_ref, w1_ref, b1_ref, w2_ref, b2_ref,
                w3_ref, b3_ref, w4_ref, b4_ref, o_ref):
    f32 = jnp.float32
    bf16 = jnp.bfloat16
    g = _unpack_bf16(g_ref[...])
    h = _unpack_bf16(h_ref[...])
    p = _unpack_bf16(p_ref[...])
    rp = jnp.concatenate(
        [jnp.dot(r_ref[b].astype(bf16), wbre_ref[...],
                 preferred_element_type=f32) for b in range(BB)],
        axis=0) + bbre_ref[...]
    A = A_ref[...]
    AT = AT_ref[...]

    zcat = jnp.concatenate([g, rp, p], axis=1).astype(bf16)
    z = jnp.tanh(jnp.dot(zcat, w1_ref[...], preferred_element_type=f32) + b1_ref[...])

    seqlen = jnp.sum(m_ref[...], axis=1)
    s = jnp.dot(A, g, preferred_element_type=f32) / seqlen[:, None]

    zproj = jnp.dot(z.astype(bf16), w2_ref[...], preferred_element_type=f32) + b2_ref[...]
    sproj = jnp.dot(s, w3_ref[...], preferred_element_type=f32) + b3_ref[...]
    gate = jax.nn.sigmoid(zproj + jnp.dot(AT, sproj, preferred_element_type=f32))
    beta = jnp.dot(gate.astype(bf16), w4_ref[...], preferred_element_type=f32) + b4_ref[...]

    S = jnp.dot(A, beta * g, preferred_element_type=f32)
    t = jnp.dot(AT, S, preferred_element_type=f32) * h
    e = jnp.exp(t)
    denom = jnp.dot(A, e, preferred_element_type=f32)
    scores = e * jnp.dot(AT, 1.0 / denom, preferred_element_type=f32)
    for b in range(BB):
        o_ref[b] = lax.slice(scores, (b * SEQL, 0), ((b + 1) * SEQL, HID))


def _fused_gating(g, h, p, review, mask, A, AT,
                  W_bre, b_bre, W1, b1, W2, b2, W3, b3, W4, b4):
    grid = BATCH // BB
    row_spec = pl.BlockSpec((ROWS, HID // 2), lambda i: (i, 0))
    full = lambda shape: pl.BlockSpec(shape, lambda i: (0, 0))
    return pl.pallas_call(
        _fused_body,
        grid=(grid,),
        in_specs=[
            row_spec,
            row_spec,
            row_spec,
            pl.BlockSpec((BB, SEQL, RDIM), lambda i: (i, 0, 0)),
            pl.BlockSpec((BB, SEQL), lambda i: (i, 0)),
            full((BB, ROWS)),
            full((ROWS, BB)),
            full((RDIM, HID)),
            full((1, HID)),
            full((3 * HID, HID)),
            full((1, HID)),
            full((HID, HID)),
            full((1, HID)),
            full((HID, HID)),
            full((1, HID)),
            full((HID, HID)),
            full((1, HID)),
        ],
        out_specs=pl.BlockSpec((BB, SEQL, HID), lambda i: (i, 0, 0)),
        out_shape=jax.ShapeDtypeStruct((BATCH, SEQL, HID), jnp.float32),
        interpret=_INTERPRET,
    )(g, h, p, review, mask, A, AT,
      W_bre, b_bre.reshape(1, HID), W1, b1.reshape(1, HID),
      W2, b2.reshape(1, HID), W3, b3.reshape(1, HID), W4, b4.reshape(1, HID))


def kernel(seq, review, mask, edge_index, emb, pos_table, Wg1, Wg2,
           W_bre, b_bre, W1, b1, W2, b2, W3, b3, W4, b4):
    src = edge_index[0]
    dst = edge_index[1]

    e = src.shape[0]
    srcp = jnp.concatenate([src, jnp.zeros((E_PAD - e,), src.dtype)]).astype(jnp.int32)
    dstp = jnp.concatenate([dst, jnp.full((E_PAD - e,), N_NODES, dst.dtype)]).astype(jnp.int32)
    srcx = jnp.stack([2 * srcp, 2 * srcp + 1]).reshape(2, E_PAD // CHUNK, CHUNK)
    dst2d = dstp.reshape(E_PAD // CHUNK, CHUNK)
    zeros = jnp.zeros((ZROWS, 128), jnp.float32)

    agg1 = _sc_segsum(emb.reshape(2 * N_NODES, 128), srcx, dst2d, zeros)
    x1 = _gcn_matmul(agg1, Wg1.astype(jnp.bfloat16), relu=True)
    agg2 = _sc_segsum(x1.reshape(2 * N_NODES, 128), srcx, dst2d, zeros)
    x2 = _gcn_matmul(agg2, Wg2.astype(jnp.bfloat16), relu=False, pack=True)

    bf16 = jnp.bfloat16

    h, g, p = _sc_gather3f(_pack_bf16(emb), x2, _pack_bf16(pos_table),
                           seq.reshape(-1).astype(jnp.int32))

    cols = jnp.arange(ROWS, dtype=jnp.int32)
    rows = jnp.arange(BB, dtype=jnp.int32)
    A = (cols[None, :] // SEQL == rows[:, None]).astype(jnp.float32)
    return _fused_gating(g, h, p, review, mask, A, A.T,
                         W_bre.astype(bf16), b_bre,
                         W1.astype(bf16), b1, W2.astype(bf16), b2,
                         W3, b3, W4.astype(bf16), b4)
